# initial kernel scaffold (unmeasured)
import jax
import jax.numpy as jnp
from jax import lax
from jax.experimental import pallas as pl
from jax.experimental.pallas import tpu as pltpu

ROWS = 4096
COLS = 1024
CHUNK = 512
MAX_CHUNKS = ROWS // CHUNK


def kernel(x, dest):
    order = jnp.argsort(dest, stable=True)
    xs = jnp.take(x.astype(jnp.bfloat16), order, axis=0)
    c0 = jnp.sum(dest == 0).astype(jnp.int32).reshape(1)

    def body(xs_ref, c0_ref, out_ref, cnt_rx_ref,
             cnt_send_sem, cnt_recv_sem, send_sems, recv_sems, copy_sems):
        my_x = lax.axis_index("x")
        my_y = lax.axis_index("y")
        peer = (my_x, 1 - my_y)

        barrier = pltpu.get_barrier_semaphore()
        pl.semaphore_signal(barrier, inc=1, device_id=peer,
                            device_id_type=pl.DeviceIdType.MESH)
        pl.semaphore_wait(barrier, 1)

        cnt_rdma = pltpu.make_async_remote_copy(
            src_ref=c0_ref, dst_ref=cnt_rx_ref,
            send_sem=cnt_send_sem, recv_sem=cnt_recv_sem,
            device_id=peer, device_id_type=pl.DeviceIdType.MESH)
        cnt_rdma.start()
        cnt_rdma.wait()

        c0_mine = c0_ref[0]
        c0_peer = cnt_rx_ref[0]
        is0 = my_y == 0
        K = jnp.where(is0, c0_mine, ROWS - c0_mine)
        S = ROWS - K
        R = jnp.where(is0, c0_peer, ROWS - c0_peer)
        src_keep = jnp.where(is0, 0, c0_mine)
        src_send = jnp.where(is0, c0_mine, 0)
        my_keep_off = jnp.where(is0, 0, R)
        my_recv_off = jnp.where(is0, K, 0)
        peer_dst_off = jnp.where(is0, 0, c0_peer)

        n_send = (S + CHUNK - 1) // CHUNK
        n_recv = (R + CHUNK - 1) // CHUNK
        n_keep = (K + CHUNK - 1) // CHUNK

        def chunk_start(k, total):
            return jnp.maximum(0, jnp.minimum(k * CHUNK, total - CHUNK))

        def send_desc(k):
            s = chunk_start(k, S)
            return pltpu.make_async_remote_copy(
                src_ref=xs_ref.at[pl.ds(src_send + s, CHUNK)],
                dst_ref=out_ref.at[pl.ds(peer_dst_off + s, CHUNK)],
                send_sem=send_sems.at[k], recv_sem=recv_sems.at[k],
                device_id=peer, device_id_type=pl.DeviceIdType.MESH)

        for k in range(MAX_CHUNKS):
            @pl.when(k < n_send)
            def _(k=k):
                send_desc(k).start()

        def keep_desc(k):
            s = chunk_start(k, K)
            return pltpu.make_async_copy(
                xs_ref.at[pl.ds(src_keep + s, CHUNK)],
                out_ref.at[pl.ds(my_keep_off + s, CHUNK)],
                copy_sems.at[k])

        for k in range(MAX_CHUNKS):
            @pl.when(k < n_keep)
            def _(k=k):
                keep_desc(k).start()
        for k in range(MAX_CHUNKS):
            @pl.when(k < n_keep)
            def _(k=k):
                keep_desc(k).wait()

        for k in range(MAX_CHUNKS):
            @pl.when(k < n_recv)
            def _(k=k):
                s = chunk_start(k, R)
                pltpu.make_async_remote_copy(
                    src_ref=xs_ref.at[pl.ds(0, CHUNK)],
                    dst_ref=out_ref.at[pl.ds(my_recv_off + s, CHUNK)],
                    send_sem=cnt_send_sem,
                    recv_sem=recv_sems.at[k],
                    device_id=peer,
                    device_id_type=pl.DeviceIdType.MESH).wait_recv()
        for k in range(MAX_CHUNKS):
            @pl.when(k < n_send)
            def _(k=k):
                send_desc(k).wait_send()

    return pl.pallas_call(
        body,
        out_shape=jax.ShapeDtypeStruct((ROWS, COLS), jnp.bfloat16),
        in_specs=[
            pl.BlockSpec(memory_space=pltpu.VMEM),
            pl.BlockSpec(memory_space=pltpu.SMEM),
        ],
        out_specs=pl.BlockSpec(memory_space=pltpu.VMEM),
        scratch_shapes=[
            pltpu.SMEM((1,), jnp.int32),
            pltpu.SemaphoreType.DMA,
            pltpu.SemaphoreType.DMA,
            pltpu.SemaphoreType.DMA((MAX_CHUNKS,)),
            pltpu.SemaphoreType.DMA((MAX_CHUNKS,)),
            pltpu.SemaphoreType.DMA((MAX_CHUNKS,)),
        ],
        compiler_params=pltpu.CompilerParams(collective_id=0),
    )(xs, c0)


# baseline (device time: 114799 ns/iter reference)
import jax
import jax.numpy as jnp
from jax import lax
from jax.experimental import pallas as pl
from jax.experimental.pallas import tpu as pltpu

ROWS = 4096
COLS = 1024
SUB = 8
CHUNK = 512
MAX_CHUNKS = ROWS // CHUNK


def kernel(x, dest):
    order = jnp.argsort(dest, stable=True)
    xs = jnp.take(x.astype(jnp.bfloat16), order, axis=0)
    xs = xs.reshape(ROWS * SUB, 128)
    c0 = jnp.sum(dest == 0).astype(jnp.int32).reshape(1)

    def body(xs_ref, c0_ref, out_ref, send_sems, recv_sems, copy_sems):
        my_x = lax.axis_index("x")
        my_y = lax.axis_index("y")
        peer = (my_x, 1 - my_y)

        barrier = pltpu.get_barrier_semaphore()
        pl.semaphore_signal(barrier, inc=1, device_id=peer,
                            device_id_type=pl.DeviceIdType.MESH)
        pl.semaphore_wait(barrier, 1)

        c0_mine = c0_ref[0]
        is0 = my_y == 0
        K = jnp.where(is0, c0_mine, ROWS - c0_mine)
        S = ROWS - K
        src_keep = jnp.where(is0, 0, c0_mine)
        src_send = jnp.where(is0, c0_mine, 0)
        my_keep_off = jnp.where(is0, 0, S)
        my_recv_off = jnp.where(is0, K, 0)
        peer_dst_off = jnp.where(is0, 0, K)

        n_comm = (S + CHUNK - 1) // CHUNK
        n_keep = (K + CHUNK - 1) // CHUNK

        def chunk_start(k, total):
            return jnp.maximum(0, jnp.minimum(k * CHUNK, total - CHUNK))

        def sl(ref, logical_off):
            return ref.at[pl.ds(pl.multiple_of(logical_off * SUB, SUB),
                                CHUNK * SUB)]

        def send_desc(k):
            s = chunk_start(k, S)
            return pltpu.make_async_remote_copy(
                src_ref=sl(xs_ref, src_send + s),
                dst_ref=sl(out_ref, peer_dst_off + s),
                send_sem=send_sems.at[k], recv_sem=recv_sems.at[k],
                device_id=peer, device_id_type=pl.DeviceIdType.MESH)

        for k in range(MAX_CHUNKS):
            @pl.when(k < n_comm)
            def _(k=k):
                send_desc(k).start()

        def keep_desc(k):
            s = chunk_start(k, K)
            return pltpu.make_async_copy(
                sl(xs_ref, src_keep + s),
                sl(out_ref, my_keep_off + s),
                copy_sems.at[k])

        for k in range(MAX_CHUNKS):
            @pl.when(k < n_keep)
            def _(k=k):
                keep_desc(k).start()
        for k in range(MAX_CHUNKS):
            @pl.when(k < n_keep)
            def _(k=k):
                keep_desc(k).wait()

        for k in range(MAX_CHUNKS):
            @pl.when(k < n_comm)
            def _(k=k):
                s = chunk_start(k, S)
                pltpu.make_async_remote_copy(
                    src_ref=sl(xs_ref, 0),
                    dst_ref=sl(out_ref, my_recv_off + s),
                    send_sem=send_sems.at[k],
                    recv_sem=recv_sems.at[k],
                    device_id=peer,
                    device_id_type=pl.DeviceIdType.MESH).wait_recv()
        for k in range(MAX_CHUNKS):
            @pl.when(k < n_comm)
            def _(k=k):
                send_desc(k).wait_send()

    out = pl.pallas_call(
        body,
        out_shape=jax.ShapeDtypeStruct((ROWS * SUB, 128), jnp.bfloat16),
        in_specs=[
            pl.BlockSpec(memory_space=pltpu.VMEM),
            pl.BlockSpec(memory_space=pltpu.SMEM),
        ],
        out_specs=pl.BlockSpec(memory_space=pltpu.VMEM),
        scratch_shapes=[
            pltpu.SemaphoreType.DMA((MAX_CHUNKS,)),
            pltpu.SemaphoreType.DMA((MAX_CHUNKS,)),
            pltpu.SemaphoreType.DMA((MAX_CHUNKS,)),
        ],
        compiler_params=pltpu.CompilerParams(collective_id=0),
    )(xs, c0)
    return out.reshape(ROWS, COLS)


# device time: 93541 ns/iter; 1.2273x vs baseline; 1.2273x over previous
import jax
import jax.numpy as jnp
from jax import lax
from jax.experimental import pallas as pl
from jax.experimental.pallas import tpu as pltpu

ROWS = 4096
COLS = 1024
SUB = 8
CHUNK = 512
MAX_CHUNKS = ROWS // CHUNK


def kernel(x, dest):
    order = jnp.argsort(dest, stable=True)
    xs = x.at[order].get(mode="promise_in_bounds", unique_indices=True)
    xs = xs.astype(jnp.bfloat16).reshape(ROWS * SUB, 128)
    c0 = jnp.sum(dest == 0).astype(jnp.int32).reshape(1)

    def body(xs_ref, c0_ref, out_ref, send_sems, recv_sems, copy_sems):
        my_x = lax.axis_index("x")
        my_y = lax.axis_index("y")
        peer = (my_x, 1 - my_y)

        barrier = pltpu.get_barrier_semaphore()
        pl.semaphore_signal(barrier, inc=1, device_id=peer,
                            device_id_type=pl.DeviceIdType.MESH)
        pl.semaphore_wait(barrier, 1)

        c0_mine = c0_ref[0]
        is0 = my_y == 0
        K = jnp.where(is0, c0_mine, ROWS - c0_mine)
        S = ROWS - K
        src_keep = jnp.where(is0, 0, c0_mine)
        src_send = jnp.where(is0, c0_mine, 0)
        my_keep_off = jnp.where(is0, 0, S)
        my_recv_off = jnp.where(is0, K, 0)
        peer_dst_off = jnp.where(is0, 0, K)

        n_comm = (S + CHUNK - 1) // CHUNK
        n_keep = (K + CHUNK - 1) // CHUNK

        def chunk_start(k, total):
            return jnp.maximum(0, jnp.minimum(k * CHUNK, total - CHUNK))

        def sl(ref, logical_off):
            return ref.at[pl.ds(pl.multiple_of(logical_off * SUB, SUB),
                                CHUNK * SUB)]

        def send_desc(k):
            s = chunk_start(k, S)
            return pltpu.make_async_remote_copy(
                src_ref=sl(xs_ref, src_send + s),
                dst_ref=sl(out_ref, peer_dst_off + s),
                send_sem=send_sems.at[k], recv_sem=recv_sems.at[k],
                device_id=peer, device_id_type=pl.DeviceIdType.MESH)

        for k in range(MAX_CHUNKS):
            @pl.when(k < n_comm)
            def _(k=k):
                send_desc(k).start()

        def keep_desc(k):
            s = chunk_start(k, K)
            return pltpu.make_async_copy(
                sl(xs_ref, src_keep + s),
                sl(out_ref, my_keep_off + s),
                copy_sems.at[k])

        for k in range(MAX_CHUNKS):
            @pl.when(k < n_keep)
            def _(k=k):
                keep_desc(k).start()
        for k in range(MAX_CHUNKS):
            @pl.when(k < n_keep)
            def _(k=k):
                keep_desc(k).wait()

        for k in range(MAX_CHUNKS):
            @pl.when(k < n_comm)
            def _(k=k):
                s = chunk_start(k, S)
                pltpu.make_async_remote_copy(
                    src_ref=sl(xs_ref, 0),
                    dst_ref=sl(out_ref, my_recv_off + s),
                    send_sem=send_sems.at[k],
                    recv_sem=recv_sems.at[k],
                    device_id=peer,
                    device_id_type=pl.DeviceIdType.MESH).wait_recv()
        for k in range(MAX_CHUNKS):
            @pl.when(k < n_comm)
            def _(k=k):
                send_desc(k).wait_send()

    out = pl.pallas_call(
        body,
        out_shape=jax.ShapeDtypeStruct((ROWS * SUB, 128), jnp.bfloat16),
        in_specs=[
            pl.BlockSpec(memory_space=pltpu.VMEM),
            pl.BlockSpec(memory_space=pltpu.SMEM),
        ],
        out_specs=pl.BlockSpec(memory_space=pltpu.VMEM),
        scratch_shapes=[
            pltpu.SemaphoreType.DMA((MAX_CHUNKS,)),
            pltpu.SemaphoreType.DMA((MAX_CHUNKS,)),
            pltpu.SemaphoreType.DMA((MAX_CHUNKS,)),
        ],
        compiler_params=pltpu.CompilerParams(collective_id=0),
    )(xs, c0)
    return out.reshape(ROWS, COLS)
